# baseline (speedup 1.0000x reference)
import jax
import jax.numpy as jnp
from jax import lax
from jax.experimental import pallas as pl
from jax.experimental.pallas import tpu as pltpu
from jax.experimental.pallas import tpu_sc as plsc

V = 10000
K = 32
F = 128
C = 3
H = 64
HE = 72
NW = 32
VPW = 320
VPAD = NW * VPW
CHUNK = 4
ROWS = CHUNK * K
NCH = VPW // CHUNK
NBUF = 2
FAST_CH = 120
SLOW_CH = 40
FAST_CORE = 0
VB = 80


def _sc_gather_body(nidx_hbm, dsq_hbm, feat_hbm, coordt_hbm, outf_hbm,
                    outq_hbm, idx_all, dsq_all, coordt_v, *bufs_and_sems):
    fbufs = bufs_and_sems[0:NBUF]
    qbufs = bufs_and_sems[NBUF:2 * NBUF]
    sems_g = bufs_and_sems[2 * NBUF:3 * NBUF]
    sems_w = bufs_and_sems[3 * NBUF:4 * NBUF]
    sems_q = bufs_and_sems[4 * NBUF:5 * NBUF]
    sid = lax.axis_index("s")
    cid = lax.axis_index("c")
    pltpu.sync_copy(coordt_hbm, coordt_v)

    zero16 = jnp.zeros((16,), jnp.float32)
    for qb in qbufs:
        for vi in range(CHUNK):
            for r in (13, 14, 15):
                for h in range(K // 16):
                    qb[vi, r, pl.ds(h * 16, 16)] = zero16

    def _run(base, nch):
        base = pl.multiple_of(base, 8)
        pltpu.sync_copy(nidx_hbm.at[pl.ds(base, nch)],
                        idx_all.at[pl.ds(0, nch)])
        pltpu.sync_copy(dsq_hbm.at[pl.ds(base, nch)],
                        dsq_all.at[pl.ds(0, nch)])

        def _qrows(j, qbuf):
            for vi in range(CHUNK):
                for h in range(K // 16):
                    sl = pl.ds(vi * K + h * 16, 16)
                    lsl = pl.ds(h * 16, 16)
                    idx16 = idx_all[j, sl]
                    e16 = jnp.exp(dsq_all[j, sl] * -10.0)
                    cc = [plsc.load_gather(coordt_v, [idx16 + c * V])
                          for c in range(C)]
                    for c in range(C):
                        for d in range(c, C):
                            p = cc[c] * cc[d] * e16
                            qbuf[vi, 3 * c + d, lsl] = p
                            if d != c:
                                qbuf[vi, 3 * d + c, lsl] = p
                        qbuf[vi, 9 + c, lsl] = cc[c] * e16
                    qbuf[vi, 12, lsl] = e16

        def _wbf_desc(j, b):
            row0 = pl.multiple_of((base + j) * CHUNK * K, 128)
            return pltpu.make_async_copy(fbufs[b],
                                         outf_hbm.at[pl.ds(row0, ROWS)],
                                         sems_w[b])

        def _wbq_desc(j, qb):
            vb = (base + j) * CHUNK
            return pltpu.make_async_copy(qbufs[qb],
                                         outq_hbm.at[pl.ds(vb, CHUNK)],
                                         sems_q[qb])

        def _gather_desc(j, b):
            return pltpu.make_async_copy(feat_hbm.at[idx_all.at[j]],
                                         fbufs[b], sems_g[b])

        def body(p, carry):
            for b in range(NBUF):
                j = NBUF * p + b

                @pl.when(p > 0)
                def _():
                    _wbf_desc(j, b).wait()
                    _wbq_desc(j, b).wait()

                _gather_desc(j, b).start()
            for b in range(NBUF):
                j = NBUF * p + b
                _qrows(j, qbufs[b])
                _gather_desc(j, b).wait()
                _wbf_desc(j, b).start()
                _wbq_desc(j, b).start()
            return carry

        nfull = nch // NBUF
        lax.fori_loop(0, nfull, body, 0)
        for jj in range(nfull * NBUF, nch):
            b = jj % NBUF
            _wbf_desc(jj, b).wait()
            _wbq_desc(jj, b).wait()
            _gather_desc(jj, b).start()
            _qrows(jj, qbufs[b])
            _gather_desc(jj, b).wait()
            _wbf_desc(jj, b).start()
            _wbq_desc(jj, b).start()
        for jj in range(nch - NBUF, nch):
            b = jj % NBUF
            _wbf_desc(jj, b).wait()
            _wbq_desc(jj, b).wait()

    @pl.when(cid == FAST_CORE)
    def _():
        _run(sid * FAST_CH, FAST_CH)

    @pl.when(cid != FAST_CORE)
    def _():
        _run(16 * FAST_CH + sid * SLOW_CH, SLOW_CH)


def _make_sc_gather():
    mesh = plsc.VectorSubcoreMesh(core_axis_name="c", subcore_axis_name="s",
                                  num_cores=2, num_subcores=16)
    return pl.kernel(
        _sc_gather_body,
        mesh=mesh,
        out_type=[
            jax.ShapeDtypeStruct((VPAD * K, F), jnp.float32),
            jax.ShapeDtypeStruct((VPAD, 16, K), jnp.float32),
        ],
        scratch_types=(
            [
                pltpu.VMEM((FAST_CH, ROWS), jnp.int32),
                pltpu.VMEM((FAST_CH, ROWS), jnp.float32),
                pltpu.VMEM((C * V,), jnp.float32),
            ]
            + [pltpu.VMEM((ROWS, F), jnp.float32) for _ in range(NBUF)]
            + [pltpu.VMEM((CHUNK, 16, K), jnp.float32) for _ in range(NBUF)]
            + [pltpu.SemaphoreType.DMA for _ in range(3 * NBUF)]
        ),
        compiler_params=pltpu.CompilerParams(needs_layout_passes=False),
    )


def _prep_weights(W0, b0, W1, b1, W2, b2):
    w0e = jnp.zeros((HE, 16), jnp.float32)
    w0e = w0e.at[:H, :C * C].set(W0.T).at[:H, 9].set(b0).at[H, 9].set(1.0)
    w1e = jnp.zeros((HE, HE), jnp.float32)
    w1e = w1e.at[:H, :H].set(W1.T).at[:H, H].set(b1).at[H, H].set(1.0)
    w2e = jnp.zeros((C * C, HE), jnp.float32)
    w2e = w2e.at[:, :H].set(W2.T).at[:, H].set(b2)
    return w0e, w1e, w2e


def _tc_body(qt_ref, nf_ref, w0_ref, w1_ref, w2_ref, out_ref):
    qt = qt_ref[...]
    nf = nf_ref[...].reshape(VB, K, F)
    m = lax.dot_general(qt, nf, (((2,), (1,)), ((0,), (0,))),
                        preferred_element_type=jnp.float32)
    m2 = jnp.concatenate([m[v] for v in range(VB)], axis=1)
    recip = 1.0 / (m2[12:13, :] + 1e-4)
    mean = m2[9:12, :] * recip
    exx = m2[0:9, :] * recip
    crows = [exx[3 * c + d:3 * c + d + 1] - mean[c:c + 1] * mean[d:d + 1]
             for c in range(C) for d in range(C)]
    crows.append(jnp.ones((1, VB * F), jnp.float32))
    crows.append(jnp.zeros((6, VB * F), jnp.float32))
    cov2 = jnp.concatenate(crows, axis=0)
    x = lax.dot_general(w0_ref[...], cov2, (((1,), (0,)), ((), ())),
                        preferred_element_type=jnp.float32)
    x = jnp.where(x > 0, x, jnp.exp(x) - 1.0)
    x = lax.dot_general(w1_ref[...], x, (((1,), (0,)), ((), ())),
                        preferred_element_type=jnp.float32)
    x = jnp.where(x > 0, x, jnp.exp(x) - 1.0)
    x3 = lax.dot_general(w2_ref[...], x, (((1,), (0,)), ((), ())),
                         preferred_element_type=jnp.float32)
    r3 = x3.reshape(C * C, VB, F)
    out_ref[...] = jnp.transpose(r3, (1, 2, 0)).reshape(VB, F * C * C)


def _make_tc_call():
    return pl.pallas_call(
        _tc_body,
        grid=(V // VB,),
        in_specs=[
            pl.BlockSpec((VB, 16, K), lambda i: (i, 0, 0)),
            pl.BlockSpec((VB * K, F), lambda i: (i, 0)),
            pl.BlockSpec((HE, 16), lambda i: (0, 0)),
            pl.BlockSpec((HE, HE), lambda i: (0, 0)),
            pl.BlockSpec((C * C, HE), lambda i: (0, 0)),
        ],
        out_specs=pl.BlockSpec((VB, F * C * C), lambda i: (i, 0)),
        out_shape=jax.ShapeDtypeStruct((V, F * C * C), jnp.float32),
    )


_tc_call = _make_tc_call()


def kernel(coordinates, distsq, features, n_idxs, W0, b0, W1, b1, W2, b2):
    nidx = n_idxs.astype(jnp.int32)
    nidx_pad = jnp.zeros((VPAD, K), jnp.int32).at[:V].set(nidx)
    nidx_pad = nidx_pad.reshape(VPAD * K // ROWS, ROWS)
    dsq_pad = jnp.zeros((VPAD, K), jnp.float32).at[:V].set(distsq)
    dsq_pad = dsq_pad.reshape(VPAD * K // ROWS, ROWS)
    coordt = coordinates.T.reshape(C * V)
    nf_g, qt_g = _make_sc_gather()(nidx_pad, dsq_pad, features, coordt)
    w0e, w1e, w2e = _prep_weights(W0, b0, W1, b1, W2, b2)
    return _tc_call(qt_g, nf_g, w0e, w1e, w2e)
